# Initial kernel scaffold; baseline (speedup 1.0000x reference)
import jax
import jax.numpy as jnp
from jax.experimental import pallas as pl


def kernel(x, Wg, W1, b1, W2, b2):
    raise NotImplementedError("write your pallas kernel here")


# speedup vs baseline: 1.9838x; 1.9838x over previous
import jax
import jax.numpy as jnp
from jax.experimental import pallas as pl
from jax.experimental.pallas import tpu as pltpu

N, D, H, E, K = 2048, 768, 768, 8, 2


def _dense_body(x_ref, wg_ref, w1_ref, b1_ref, w2_ref, b2_ref, y_ref):
    e = pl.program_id(0)
    x = x_ref[...]

    logits = jnp.dot(x, wg_ref[...], preferred_element_type=jnp.float32)
    cols = jax.lax.broadcasted_iota(jnp.int32, logits.shape, 1)
    m1 = jnp.max(logits, axis=1, keepdims=True)
    a1 = jnp.argmax(logits, axis=1).reshape(-1, 1)
    neg = jnp.full_like(logits, -jnp.inf)
    masked = jnp.where(cols == a1, neg, logits)
    m2 = jnp.max(masked, axis=1, keepdims=True)
    a2 = jnp.argmax(masked, axis=1).reshape(-1, 1)
    t = jnp.exp(m2 - m1)
    w1g = 1.0 / (1.0 + t)
    w2g = t / (1.0 + t)
    gate = jnp.where(a1 == e, w1g, jnp.where(a2 == e, w2g, 0.0))

    h = jnp.maximum(
        jnp.dot(x, w1_ref[0], preferred_element_type=jnp.float32) + b1_ref[0], 0.0)
    o = jnp.dot(h, w2_ref[0], preferred_element_type=jnp.float32) + b2_ref[0]
    contrib = o * gate

    @pl.when(e == 0)
    def _():
        y_ref[...] = contrib

    @pl.when(e != 0)
    def _():
        y_ref[...] = y_ref[...] + contrib


def kernel(x, Wg, W1, b1, W2, b2):
    return pl.pallas_call(
        _dense_body,
        grid=(E,),
        in_specs=[
            pl.BlockSpec((N, D), lambda e: (0, 0)),
            pl.BlockSpec((D, E), lambda e: (0, 0)),
            pl.BlockSpec((1, D, H), lambda e: (e, 0, 0)),
            pl.BlockSpec((1, 1, H), lambda e: (e, 0, 0)),
            pl.BlockSpec((1, H, D), lambda e: (e, 0, 0)),
            pl.BlockSpec((1, 1, D), lambda e: (e, 0, 0)),
        ],
        out_specs=pl.BlockSpec((N, D), lambda e: (0, 0)),
        out_shape=jax.ShapeDtypeStruct((N, D), jnp.float32),
    )(x, Wg, W1, b1.reshape(E, 1, H), W2, b2.reshape(E, 1, D))
